# baseline (speedup 1.0000x reference)
import functools

import jax
import jax.numpy as jnp
import numpy as np
from jax import lax
from jax.experimental import pallas as pl
from jax.experimental.pallas import tpu as pltpu
from jax.experimental.pallas import tpu_sc as plsc

B = 4
KTOP = 64
C = 16
N = 8192
F = 32
Q = KTOP * C
QTOT = B * Q
KNN = 32
ROW_OUT = KNN * (3 + F)

NW = 32
QPW = QTOT // NW
WPB = Q // QPW

_INF = np.float32(np.inf)


def _tc_d2_body(q_ref, t_ref, d2_ref):
    q = q_ref[0]
    t = t_ref[0]
    q2 = jnp.sum(q * q, axis=1)
    r2 = jnp.sum(t * t, axis=1)
    prod = lax.dot_general(q, t, (((1,), (1,)), ((), ())),
                           preferred_element_type=jnp.float32)
    d2_ref[0] = q2[:, None] + r2[None, :] - 2.0 * prod


def _tc_d2(qp, tp):
    NT = 512
    return pl.pallas_call(
        _tc_d2_body,
        grid=(B, N // NT),
        in_specs=[
            pl.BlockSpec((1, Q, 3), lambda b, n: (b, 0, 0)),
            pl.BlockSpec((1, NT, 3), lambda b, n: (b, n, 0)),
        ],
        out_specs=pl.BlockSpec((1, Q, NT), lambda b, n: (b, 0, n)),
        out_shape=jax.ShapeDtypeStruct((B, Q, N), jnp.float32),
    )(qp, tp)


def _any_below(v, m_splat):
    cnt = plsc.all_reduce_population_count(v < m_splat)
    return cnt[0] > 0


def _merge_topk(v, iv, carry):
    lo, hi, ilo, ihi, m = carry
    mask = v < m
    cv = jnp.where(mask, v, _INF)
    ci = jnp.where(mask, iv, 0)
    cv, ci = plsc.sort_key_val(cv, ci)
    rc = jnp.flip(cv)
    ric = jnp.flip(ci)
    le = lo <= rc
    a = jnp.minimum(lo, rc)
    ia = jnp.where(le, ilo, ric)
    bm = jnp.maximum(lo, rc)
    ib = jnp.where(le, ric, ilo)
    lo2, ilo2 = plsc.sort_key_val(a, ia)
    bs, ibs = plsc.sort_key_val(bm, ib)
    rb = jnp.flip(bs)
    rib = jnp.flip(ibs)
    le2 = hi <= rb
    c2 = jnp.minimum(hi, rb)
    ic2 = jnp.where(le2, ihi, rib)
    hi2, ihi2 = plsc.sort_key_val(c2, ic2)
    m2 = jnp.full((16,), hi2[15], jnp.float32)
    return (lo2, hi2, ilo2, ihi2, m2)


def _tie_fix(lo, hi, ilo, ihi, iota):
    even_mask = (iota & 1) == 0
    odd_mask = (iota & 1) == 1
    pe = iota ^ 1
    po = jnp.where((iota == 0) | (iota == 15), iota,
                   jnp.where(odd_mask, iota + 1, iota - 1))

    def fix(v, i, pv, pi, take_lower):
        eq = v == pv
        return jnp.where(eq, jnp.where(take_lower, jnp.minimum(i, pi),
                                       jnp.maximum(i, pi)), i)

    def g(x, p):
        return x.at[p].get(mode="promise_in_bounds")

    for _ in range(2):
        ilo = fix(lo, ilo, g(lo, pe), g(ilo, pe), even_mask)
        ihi = fix(hi, ihi, g(hi, pe), g(ihi, pe), even_mask)
        is15 = iota == 15
        is0 = iota == 0
        ilo_prev = ilo
        pv = jnp.where(is15, jnp.full((16,), hi[0], jnp.float32),
                       g(lo, po))
        pi = jnp.where(is15, jnp.full((16,), ihi[0], jnp.int32),
                       g(ilo, po))
        ilo = fix(lo, ilo, pv, pi, odd_mask)
        pv = jnp.where(is0, jnp.full((16,), lo[15], jnp.float32),
                       g(hi, po))
        pi = jnp.where(is0, jnp.full((16,), ilo_prev[15], jnp.int32),
                       g(ihi, po))
        ihi = fix(hi, ihi, pv, pi, odd_mask)
    return ilo, ihi


def _heron_sqrt(x):
    i = lax.bitcast_convert_type(x, jnp.int32)
    i = (i >> 1) + np.int32(0x1FBD1DF5)
    y = lax.bitcast_convert_type(i, jnp.float32)
    for _ in range(4):
        y = 0.5 * (y + x / y)
    return y


def _make_sc_kernel():
    mesh = plsc.VectorSubcoreMesh(core_axis_name="c", subcore_axis_name="s")

    @functools.partial(
        pl.kernel,
        out_type=jax.ShapeDtypeStruct((QTOT * ROW_OUT,), jnp.float32),
        mesh=mesh,
        compiler_params=pltpu.CompilerParams(needs_layout_passes=False,
                                             use_tc_tiling_on_sc=False),
        scratch_types=[
            pltpu.VMEM((N,), jnp.float32),
            pltpu.VMEM((N,), jnp.float32),
            pltpu.VMEM((N,), jnp.float32),
            pltpu.VMEM((N,), jnp.float32),
            pltpu.VMEM((N,), jnp.float32),
            pltpu.VMEM((QPW * 3 + 16,), jnp.float32),
            pltpu.VMEM((KNN, F), jnp.float32),
            pltpu.VMEM((ROW_OUT,), jnp.float32),
            pltpu.SemaphoreType.DMA,
            pltpu.SemaphoreType.DMA,
        ],
    )
    def sc_kernel(d2_hbm, x_hbm, y_hbm, z_hbm, q_hbm, feat_hbm, out_hbm,
                  rowbuf0, rowbuf1, xbuf, ybuf, zbuf, qbuf, featbuf, outbuf,
                  rsem, fsem):
        cid = lax.axis_index("c")
        sid = lax.axis_index("s")
        wid = cid * 16 + sid
        b = wid // WPB
        q0 = wid * QPW

        pltpu.sync_copy(x_hbm.at[pl.ds(b * N, N)], xbuf)
        pltpu.sync_copy(y_hbm.at[pl.ds(b * N, N)], ybuf)
        pltpu.sync_copy(z_hbm.at[pl.ds(b * N, N)], zbuf)
        pltpu.sync_copy(q_hbm.at[pl.ds(q0 * 3, QPW * 3)],
                        qbuf.at[pl.ds(0, QPW * 3)])

        iota = lax.iota(jnp.int32, 16)
        offs_lo = iota * (3 + F)
        offs_hi = offs_lo + 16 * (3 + F)

        pltpu.async_copy(d2_hbm.at[pl.ds(q0 * N, N)], rowbuf0, rsem)

        def process(iq, row, nextrow):
            gq = q0 + iq
            pltpu.make_async_copy(d2_hbm.at[pl.ds(gq * N, N)], row,
                                  rsem).wait()

            @pl.when(iq < QPW - 1)
            def _prefetch():
                pltpu.async_copy(d2_hbm.at[pl.ds((gq + 1) * N, N)],
                                 nextrow, rsem)

            def group_body(g, carry):
                base = g * 16
                vj = row[pl.ds(base, 16)]

                def do_merge(c2):
                    return _merge_topk(vj, iota + base, c2)

                return lax.cond(_any_below(vj, carry[4]), do_merge,
                                lambda c2: c2, carry)

            zero_i = jnp.zeros((16,), jnp.int32)
            inf_v = jnp.full((16,), _INF, jnp.float32)
            lo, hi, ilo, ihi, _m = lax.fori_loop(
                0, N // 16, group_body,
                (inf_v, inf_v, zero_i, zero_i, inf_v))
            ilo, ihi = _tie_fix(lo, hi, ilo, ihi, iota)

            dlo = _heron_sqrt(jnp.maximum(lo, 1e-12))
            dhi = _heron_sqrt(jnp.maximum(hi, 1e-12))
            dsum = dlo + dhi
            for k in (1, 2, 4, 8):
                dsum = dsum + dsum.at[iota ^ k].get(
                    mode="promise_in_bounds")
            wlo = dlo / dsum
            whi = dhi / dsum

            gidx_lo = ilo + b * N
            gidx_hi = ihi + b * N
            fcp1 = pltpu.async_copy(feat_hbm.at[gidx_lo],
                                    featbuf.at[pl.ds(0, 16)], fsem)
            fcp2 = pltpu.async_copy(feat_hbm.at[gidx_hi],
                                    featbuf.at[pl.ds(16, 16)], fsem)

            qv = qbuf[pl.ds(3 * iq, 16)]
            qx = qv[0]
            qy = qv[1]
            qz = qv[2]
            dxl = plsc.load_gather(xbuf, [ilo]) - qx
            dyl = plsc.load_gather(ybuf, [ilo]) - qy
            dzl = plsc.load_gather(zbuf, [ilo]) - qz
            dxh = plsc.load_gather(xbuf, [ihi]) - qx
            dyh = plsc.load_gather(ybuf, [ihi]) - qy
            dzh = plsc.load_gather(zbuf, [ihi]) - qz
            plsc.store_scatter(outbuf, [offs_lo], dxl)
            plsc.store_scatter(outbuf, [offs_lo + 1], dyl)
            plsc.store_scatter(outbuf, [offs_lo + 2], dzl)
            plsc.store_scatter(outbuf, [offs_hi], dxh)
            plsc.store_scatter(outbuf, [offs_hi + 1], dyh)
            plsc.store_scatter(outbuf, [offs_hi + 2], dzh)

            fcp1.wait()
            fcp2.wait()
            for j in range(KNN):
                fl = featbuf[j, pl.ds(0, 16)]
                fh = featbuf[j, pl.ds(16, 16)]
                o = j * (3 + F) + 3
                outbuf[pl.ds(o, 16)] = fl * wlo
                outbuf[pl.ds(o + 16, 16)] = fh * whi

            pltpu.sync_copy(outbuf, out_hbm.at[pl.ds(gq * ROW_OUT, ROW_OUT)])

        def pair_body(it, carry):
            process(it * 2, rowbuf0, rowbuf1)
            process(it * 2 + 1, rowbuf1, rowbuf0)
            return carry

        lax.fori_loop(0, QPW // 2, pair_body, 0)

    return sc_kernel


_sc_kernel = _make_sc_kernel()


def kernel(candidate_pts, src_keypts, tgt_pts_xyz, tgt_deep_feat_pts):
    del src_keypts
    d2 = _tc_d2(candidate_pts.reshape(B, Q, 3), tgt_pts_xyz)

    tt = tgt_pts_xyz.transpose(0, 2, 1)
    x = tt[:, 0, :].reshape(-1)
    y = tt[:, 1, :].reshape(-1)
    z = tt[:, 2, :].reshape(-1)
    q_flat = candidate_pts.reshape(-1)
    feat_flat = tgt_deep_feat_pts.reshape(B * N, F)

    out_flat = _sc_kernel(d2.reshape(-1), x, y, z, q_flat, feat_flat)
    return out_flat.reshape(B, KTOP, C, KNN, 3 + F)


# speedup vs baseline: 1.6931x; 1.6931x over previous
import functools

import jax
import jax.numpy as jnp
import numpy as np
from jax import lax
from jax.experimental import pallas as pl
from jax.experimental.pallas import tpu as pltpu
from jax.experimental.pallas import tpu_sc as plsc

B = 4
KTOP = 64
C = 16
N = 8192
F = 32
Q = KTOP * C
QTOT = B * Q
KNN = 32
ROW_OUT = KNN * (3 + F)

NW = 32
QPW = QTOT // NW
WPB = Q // QPW

_INF = np.float32(np.inf)


def _tc_d2_body(q_ref, t_ref, d2_ref):
    q = q_ref[0]
    t = t_ref[0]
    q2 = jnp.sum(q * q, axis=1)
    r2 = jnp.sum(t * t, axis=1)
    prod = lax.dot_general(q, t, (((1,), (1,)), ((), ())),
                           preferred_element_type=jnp.float32)
    d2_ref[0] = q2[:, None] + r2[None, :] - 2.0 * prod


def _tc_d2(qp, tp):
    NT = 512
    return pl.pallas_call(
        _tc_d2_body,
        grid=(B, N // NT),
        in_specs=[
            pl.BlockSpec((1, Q, 3), lambda b, n: (b, 0, 0)),
            pl.BlockSpec((1, NT, 3), lambda b, n: (b, n, 0)),
        ],
        out_specs=pl.BlockSpec((1, Q, NT), lambda b, n: (b, 0, n)),
        out_shape=jax.ShapeDtypeStruct((B, Q, N), jnp.float32),
    )(qp, tp)


def _any_below(v, m_splat):
    cnt = plsc.all_reduce_population_count(v < m_splat)
    return cnt[0] > 0


def _merge_topk(v, iv, carry):
    lo, hi, ilo, ihi, m = carry
    mask = v < m
    cv = jnp.where(mask, v, _INF)
    ci = jnp.where(mask, iv, 0)
    cv, ci = plsc.sort_key_val(cv, ci)
    rc = jnp.flip(cv)
    ric = jnp.flip(ci)
    le = lo <= rc
    a = jnp.minimum(lo, rc)
    ia = jnp.where(le, ilo, ric)
    bm = jnp.maximum(lo, rc)
    ib = jnp.where(le, ric, ilo)
    lo2, ilo2 = plsc.sort_key_val(a, ia)
    bs, ibs = plsc.sort_key_val(bm, ib)
    rb = jnp.flip(bs)
    rib = jnp.flip(ibs)
    le2 = hi <= rb
    c2 = jnp.minimum(hi, rb)
    ic2 = jnp.where(le2, ihi, rib)
    hi2, ihi2 = plsc.sort_key_val(c2, ic2)
    m2 = jnp.full((16,), hi2[15], jnp.float32)
    return (lo2, hi2, ilo2, ihi2, m2)


def _tie_fix(lo, hi, ilo, ihi, iota):
    even_mask = (iota & 1) == 0
    odd_mask = (iota & 1) == 1
    pe = iota ^ 1
    po = jnp.where((iota == 0) | (iota == 15), iota,
                   jnp.where(odd_mask, iota + 1, iota - 1))

    def fix(v, i, pv, pi, take_lower):
        eq = v == pv
        return jnp.where(eq, jnp.where(take_lower, jnp.minimum(i, pi),
                                       jnp.maximum(i, pi)), i)

    def g(x, p):
        return x.at[p].get(mode="promise_in_bounds")

    for _ in range(2):
        ilo = fix(lo, ilo, g(lo, pe), g(ilo, pe), even_mask)
        ihi = fix(hi, ihi, g(hi, pe), g(ihi, pe), even_mask)
        is15 = iota == 15
        is0 = iota == 0
        ilo_prev = ilo
        pv = jnp.where(is15, jnp.full((16,), hi[0], jnp.float32),
                       g(lo, po))
        pi = jnp.where(is15, jnp.full((16,), ihi[0], jnp.int32),
                       g(ilo, po))
        ilo = fix(lo, ilo, pv, pi, odd_mask)
        pv = jnp.where(is0, jnp.full((16,), lo[15], jnp.float32),
                       g(hi, po))
        pi = jnp.where(is0, jnp.full((16,), ilo_prev[15], jnp.int32),
                       g(ihi, po))
        ihi = fix(hi, ihi, pv, pi, odd_mask)
    return ilo, ihi


def _heron_sqrt(x):
    i = lax.bitcast_convert_type(x, jnp.int32)
    i = (i >> 1) + np.int32(0x1FBD1DF5)
    y = lax.bitcast_convert_type(i, jnp.float32)
    for _ in range(4):
        y = 0.5 * (y + x / y)
    return y


def _make_sc_kernel():
    mesh = plsc.VectorSubcoreMesh(core_axis_name="c", subcore_axis_name="s")

    @functools.partial(
        pl.kernel,
        out_type=jax.ShapeDtypeStruct((QTOT * ROW_OUT,), jnp.float32),
        mesh=mesh,
        compiler_params=pltpu.CompilerParams(needs_layout_passes=False,
                                             use_tc_tiling_on_sc=False),
        scratch_types=[
            pltpu.VMEM((N,), jnp.float32),
            pltpu.VMEM((N,), jnp.float32),
            pltpu.VMEM((N,), jnp.float32),
            pltpu.VMEM((N,), jnp.float32),
            pltpu.VMEM((N,), jnp.float32),
            pltpu.VMEM((QPW * 3 + 16,), jnp.float32),
            pltpu.VMEM((KNN, F), jnp.float32),
            pltpu.VMEM((ROW_OUT,), jnp.float32),
            pltpu.SemaphoreType.DMA,
            pltpu.SemaphoreType.DMA,
        ],
    )
    def sc_kernel(d2_hbm, x_hbm, y_hbm, z_hbm, q_hbm, feat_hbm, out_hbm,
                  rowbuf0, rowbuf1, xbuf, ybuf, zbuf, qbuf, featbuf, outbuf,
                  rsem, fsem):
        cid = lax.axis_index("c")
        sid = lax.axis_index("s")
        wid = cid * 16 + sid
        b = wid // WPB
        q0 = wid * QPW

        pltpu.sync_copy(x_hbm.at[pl.ds(b * N, N)], xbuf)
        pltpu.sync_copy(y_hbm.at[pl.ds(b * N, N)], ybuf)
        pltpu.sync_copy(z_hbm.at[pl.ds(b * N, N)], zbuf)
        pltpu.sync_copy(q_hbm.at[pl.ds(q0 * 3, QPW * 3)],
                        qbuf.at[pl.ds(0, QPW * 3)])

        iota = lax.iota(jnp.int32, 16)
        offs_lo = iota * (3 + F)
        offs_hi = offs_lo + 16 * (3 + F)

        pltpu.async_copy(d2_hbm.at[pl.ds(q0 * N, N)], rowbuf0, rsem)

        def process(iq, row, nextrow):
            gq = q0 + iq
            pltpu.make_async_copy(d2_hbm.at[pl.ds(gq * N, N)], row,
                                  rsem).wait()

            @pl.when(iq < QPW - 1)
            def _prefetch():
                pltpu.async_copy(d2_hbm.at[pl.ds((gq + 1) * N, N)],
                                 nextrow, rsem)

            zero_i = jnp.zeros((16,), jnp.int32)
            inf_v = jnp.full((16,), _INF, jnp.float32)

            @plsc.parallel_loop(0, N // 32, unroll=4,
                                carry=(inf_v, inf_v, zero_i, zero_i, inf_v))
            def selection(g, carry):
                base = g * 32
                vs = [row[pl.ds(base + 16 * j, 16)] for j in range(2)]
                gm = jnp.minimum(vs[0], vs[1])

                def hit(cr):
                    for j in range(2):
                        vj = vs[j]
                        ivj = iota + (base + 16 * j)

                        def do_merge(c2, vj=vj, ivj=ivj):
                            return _merge_topk(vj, ivj, c2)

                        cr = lax.cond(_any_below(vj, cr[4]), do_merge,
                                      lambda c2: c2, cr)
                    return cr

                return lax.cond(_any_below(gm, carry[4]), hit,
                                lambda c2: c2, carry)

            lo, hi, ilo, ihi, _m = selection
            ilo, ihi = _tie_fix(lo, hi, ilo, ihi, iota)

            dlo = _heron_sqrt(jnp.maximum(lo, 1e-12))
            dhi = _heron_sqrt(jnp.maximum(hi, 1e-12))
            dsum = dlo + dhi
            for k in (1, 2, 4, 8):
                dsum = dsum + dsum.at[iota ^ k].get(
                    mode="promise_in_bounds")
            wlo = dlo / dsum
            whi = dhi / dsum

            gidx_lo = ilo + b * N
            gidx_hi = ihi + b * N
            fcp1 = pltpu.async_copy(feat_hbm.at[gidx_lo],
                                    featbuf.at[pl.ds(0, 16)], fsem)
            fcp2 = pltpu.async_copy(feat_hbm.at[gidx_hi],
                                    featbuf.at[pl.ds(16, 16)], fsem)

            qv = qbuf[pl.ds(3 * iq, 16)]
            qx = qv[0]
            qy = qv[1]
            qz = qv[2]
            dxl = plsc.load_gather(xbuf, [ilo]) - qx
            dyl = plsc.load_gather(ybuf, [ilo]) - qy
            dzl = plsc.load_gather(zbuf, [ilo]) - qz
            dxh = plsc.load_gather(xbuf, [ihi]) - qx
            dyh = plsc.load_gather(ybuf, [ihi]) - qy
            dzh = plsc.load_gather(zbuf, [ihi]) - qz
            plsc.store_scatter(outbuf, [offs_lo], dxl)
            plsc.store_scatter(outbuf, [offs_lo + 1], dyl)
            plsc.store_scatter(outbuf, [offs_lo + 2], dzl)
            plsc.store_scatter(outbuf, [offs_hi], dxh)
            plsc.store_scatter(outbuf, [offs_hi + 1], dyh)
            plsc.store_scatter(outbuf, [offs_hi + 2], dzh)

            fcp1.wait()
            fcp2.wait()
            for j in range(KNN):
                fl = featbuf[j, pl.ds(0, 16)]
                fh = featbuf[j, pl.ds(16, 16)]
                o = j * (3 + F) + 3
                outbuf[pl.ds(o, 16)] = fl * wlo
                outbuf[pl.ds(o + 16, 16)] = fh * whi

            pltpu.sync_copy(outbuf, out_hbm.at[pl.ds(gq * ROW_OUT, ROW_OUT)])

        def pair_body(it, carry):
            process(it * 2, rowbuf0, rowbuf1)
            process(it * 2 + 1, rowbuf1, rowbuf0)
            return carry

        lax.fori_loop(0, QPW // 2, pair_body, 0)

    return sc_kernel


_sc_kernel = _make_sc_kernel()


def kernel(candidate_pts, src_keypts, tgt_pts_xyz, tgt_deep_feat_pts):
    del src_keypts
    d2 = _tc_d2(candidate_pts.reshape(B, Q, 3), tgt_pts_xyz)

    tt = tgt_pts_xyz.transpose(0, 2, 1)
    x = tt[:, 0, :].reshape(-1)
    y = tt[:, 1, :].reshape(-1)
    z = tt[:, 2, :].reshape(-1)
    q_flat = candidate_pts.reshape(-1)
    feat_flat = tgt_deep_feat_pts.reshape(B * N, F)

    out_flat = _sc_kernel(d2.reshape(-1), x, y, z, q_flat, feat_flat)
    return out_flat.reshape(B, KTOP, C, KNN, 3 + F)
